# baseline (device time: 10334 ns/iter reference)
import jax
import jax.numpy as jnp
from jax import lax
from jax.experimental import pallas as pl
from jax.experimental.pallas import tpu as pltpu

N_DEV = 8


def kernel(x, w_mat):
    m_per, k = x.shape
    _, n = w_mat.shape
    blk = n // N_DEV

    def body(x_hbm, w_hbm, out_ref, x_ref, w_ref, ysend_ref,
             load_sems, store_sem, send_sems, recv_sems):
        my = lax.axis_index("i")

        barrier_sem = pltpu.get_barrier_semaphore()
        for d in range(1, N_DEV):
            pl.semaphore_signal(
                barrier_sem, inc=1,
                device_id=((my + d) % N_DEV,),
                device_id_type=pl.DeviceIdType.MESH,
            )

        xcp = pltpu.make_async_copy(x_hbm, x_ref, load_sems.at[0])
        wcp = pltpu.make_async_copy(w_hbm, w_ref, load_sems.at[1])
        xcp.start()
        wcp.start()
        xcp.wait()
        wcp.wait()

        y = jnp.dot(
            x_ref[...].astype(jnp.bfloat16),
            w_ref[...].astype(jnp.bfloat16),
            preferred_element_type=jnp.float32,
        )
        yb = jnp.maximum(y, 0.0).astype(jnp.bfloat16)
        for j in range(N_DEV):
            ysend_ref[j] = yb[:, j * blk:(j + 1) * blk]

        pl.semaphore_wait(barrier_sem, N_DEV - 1)

        sends = []
        for d in range(1, N_DEV):
            tgt = (my + d) % N_DEV
            rdma = pltpu.make_async_remote_copy(
                src_ref=ysend_ref.at[tgt],
                dst_ref=out_ref.at[pl.ds(my * m_per, m_per), :],
                send_sem=send_sems.at[d - 1],
                recv_sem=recv_sems.at[my],
                device_id=(tgt,),
                device_id_type=pl.DeviceIdType.MESH,
            )
            rdma.start()
            sends.append(rdma)

        own = pltpu.make_async_copy(
            ysend_ref.at[my],
            out_ref.at[pl.ds(my * m_per, m_per), :],
            store_sem,
        )
        own.start()

        for d in range(1, N_DEV):
            src = (my - d) % N_DEV
            recv = pltpu.make_async_remote_copy(
                src_ref=ysend_ref.at[src],
                dst_ref=out_ref.at[pl.ds(src * m_per, m_per), :],
                send_sem=send_sems.at[d - 1],
                recv_sem=recv_sems.at[src],
                device_id=(src,),
                device_id_type=pl.DeviceIdType.MESH,
            )
            recv.wait_recv()

        own.wait()
        for rdma in sends:
            rdma.wait_send()

    out_shape = jax.ShapeDtypeStruct((N_DEV * m_per, blk), jnp.bfloat16)
    return pl.pallas_call(
        body,
        out_shape=out_shape,
        in_specs=[
            pl.BlockSpec(memory_space=pl.ANY),
            pl.BlockSpec(memory_space=pl.ANY),
        ],
        out_specs=pl.BlockSpec(memory_space=pl.ANY),
        scratch_shapes=[
            pltpu.VMEM((m_per, k), jnp.float32),
            pltpu.VMEM((k, n), jnp.float32),
            pltpu.VMEM((N_DEV, m_per, blk), jnp.bfloat16),
            pltpu.SemaphoreType.DMA((2,)),
            pltpu.SemaphoreType.DMA,
            pltpu.SemaphoreType.DMA((N_DEV - 1,)),
            pltpu.SemaphoreType.DMA((N_DEV,)),
        ],
        compiler_params=pltpu.CompilerParams(collective_id=0),
    )(
        pltpu.with_memory_space_constraint(x, pltpu.MemorySpace.HBM),
        pltpu.with_memory_space_constraint(w_mat, pltpu.MemorySpace.HBM),
    )


# device time: 9383 ns/iter; 1.1014x vs baseline; 1.1014x over previous
import jax
import jax.numpy as jnp
from jax import lax
from jax.experimental import pallas as pl
from jax.experimental.pallas import tpu as pltpu

N_DEV = 8
N_HALF = N_DEV // 2


def kernel(x, w_mat):
    m_per, k = x.shape
    _, n = w_mat.shape
    blk = n // N_DEV
    half = n // 2

    def body(x_hbm, w_hbm, out_ref, x_ref, w_ref, ysend_ref,
             load_sems, store_sem, send_sems, recv_sems):
        my = lax.axis_index("i")

        barrier_sem = pltpu.get_barrier_semaphore()
        for d in range(1, N_DEV):
            pl.semaphore_signal(
                barrier_sem, inc=1,
                device_id=((my + d) % N_DEV,),
                device_id_type=pl.DeviceIdType.MESH,
            )

        xcp = pltpu.make_async_copy(x_hbm, x_ref, load_sems.at[0])
        wcp0 = pltpu.make_async_copy(
            w_hbm.at[:, pl.ds(0, half)], w_ref.at[:, pl.ds(0, half)],
            load_sems.at[1],
        )
        wcp1 = pltpu.make_async_copy(
            w_hbm.at[:, pl.ds(half, half)], w_ref.at[:, pl.ds(half, half)],
            load_sems.at[2],
        )
        xcp.start()
        wcp0.start()
        wcp1.start()
        xcp.wait()
        xb = x_ref[...].astype(jnp.bfloat16)

        sends = []

        def compute_half(h):
            y = jnp.dot(
                xb,
                w_ref[:, h * half:(h + 1) * half].astype(jnp.bfloat16),
                preferred_element_type=jnp.float32,
            )
            yb = jnp.maximum(y, 0.0).astype(jnp.bfloat16)
            for j in range(N_HALF):
                ysend_ref[h * N_HALF + j] = yb[:, j * blk:(j + 1) * blk]

        def send_half(h):
            lo, hi = h * N_HALF, (h + 1) * N_HALF
            for d in range(1, N_DEV):
                tgt = (my + d) % N_DEV
                rdma = pltpu.make_async_remote_copy(
                    src_ref=ysend_ref.at[tgt],
                    dst_ref=out_ref.at[pl.ds(my * m_per, m_per), :],
                    send_sem=send_sems.at[d - 1],
                    recv_sem=recv_sems.at[my],
                    device_id=(tgt,),
                    device_id_type=pl.DeviceIdType.MESH,
                )

                @pl.when(jnp.logical_and(tgt >= lo, tgt < hi))
                def _():
                    rdma.start()

                if h == 0:
                    sends.append(rdma)

        wcp0.wait()
        compute_half(0)
        pl.semaphore_wait(barrier_sem, N_DEV - 1)
        send_half(0)
        wcp1.wait()
        compute_half(1)
        send_half(1)

        own = pltpu.make_async_copy(
            ysend_ref.at[my],
            out_ref.at[pl.ds(my * m_per, m_per), :],
            store_sem,
        )
        own.start()

        for d in range(1, N_DEV):
            src = (my - d) % N_DEV
            recv = pltpu.make_async_remote_copy(
                src_ref=ysend_ref.at[src],
                dst_ref=out_ref.at[pl.ds(src * m_per, m_per), :],
                send_sem=send_sems.at[d - 1],
                recv_sem=recv_sems.at[src],
                device_id=(src,),
                device_id_type=pl.DeviceIdType.MESH,
            )
            recv.wait_recv()

        own.wait()
        for rdma in sends:
            rdma.wait_send()

    out_shape = jax.ShapeDtypeStruct((N_DEV * m_per, blk), jnp.bfloat16)
    return pl.pallas_call(
        body,
        out_shape=out_shape,
        in_specs=[
            pl.BlockSpec(memory_space=pl.ANY),
            pl.BlockSpec(memory_space=pl.ANY),
        ],
        out_specs=pl.BlockSpec(memory_space=pl.ANY),
        scratch_shapes=[
            pltpu.VMEM((m_per, k), jnp.float32),
            pltpu.VMEM((k, n), jnp.float32),
            pltpu.VMEM((N_DEV, m_per, blk), jnp.bfloat16),
            pltpu.SemaphoreType.DMA((3,)),
            pltpu.SemaphoreType.DMA,
            pltpu.SemaphoreType.DMA((N_DEV - 1,)),
            pltpu.SemaphoreType.DMA((N_DEV,)),
        ],
        compiler_params=pltpu.CompilerParams(collective_id=0),
    )(
        pltpu.with_memory_space_constraint(x, pltpu.MemorySpace.HBM),
        pltpu.with_memory_space_constraint(w_mat, pltpu.MemorySpace.HBM),
    )


# device time: 8980 ns/iter; 1.1508x vs baseline; 1.0449x over previous
import jax
import jax.numpy as jnp
from jax import lax
from jax.experimental import pallas as pl
from jax.experimental.pallas import tpu as pltpu

N_DEV = 8
N_CHUNKS = 4
BLKS_PER_CHUNK = N_DEV // N_CHUNKS


def kernel(x, w_mat):
    m_per, k = x.shape
    _, n = w_mat.shape
    blk = n // N_DEV
    chunk = n // N_CHUNKS

    def body(x_hbm, w_hbm, out_ref, x_ref, w_ref, ysend_ref,
             load_sems, store_sem, send_sems, recv_sems):
        my = lax.axis_index("i")

        barrier_sem = pltpu.get_barrier_semaphore()
        for d in range(1, N_DEV):
            pl.semaphore_signal(
                barrier_sem, inc=1,
                device_id=((my + d) % N_DEV,),
                device_id_type=pl.DeviceIdType.MESH,
            )

        xcp = pltpu.make_async_copy(x_hbm, x_ref, load_sems.at[0])
        xcp.start()
        wcps = []
        for c in range(N_CHUNKS):
            wcp = pltpu.make_async_copy(
                w_hbm.at[:, pl.ds(c * chunk, chunk)],
                w_ref.at[:, pl.ds(c * chunk, chunk)],
                load_sems.at[c + 1],
            )
            wcp.start()
            wcps.append(wcp)
        xcp.wait()
        xb = x_ref[...].astype(jnp.bfloat16)

        sends = []

        def compute_chunk(c):
            y = jnp.dot(
                xb,
                w_ref[:, c * chunk:(c + 1) * chunk].astype(jnp.bfloat16),
                preferred_element_type=jnp.float32,
            )
            yb = jnp.maximum(y, 0.0).astype(jnp.bfloat16)
            for j in range(BLKS_PER_CHUNK):
                ysend_ref[c * BLKS_PER_CHUNK + j] = yb[:, j * blk:(j + 1) * blk]

        def send_chunk(c):
            lo, hi = c * BLKS_PER_CHUNK, (c + 1) * BLKS_PER_CHUNK
            for d in range(1, N_DEV):
                tgt = (my + d) % N_DEV
                rdma = pltpu.make_async_remote_copy(
                    src_ref=ysend_ref.at[tgt],
                    dst_ref=out_ref.at[pl.ds(my * m_per, m_per), :],
                    send_sem=send_sems.at[d - 1],
                    recv_sem=recv_sems.at[my],
                    device_id=(tgt,),
                    device_id_type=pl.DeviceIdType.MESH,
                )

                @pl.when(jnp.logical_and(tgt >= lo, tgt < hi))
                def _():
                    rdma.start()

                if c == 0:
                    sends.append(rdma)

        for c in range(N_CHUNKS):
            wcps[c].wait()
            compute_chunk(c)
            if c == 0:
                pl.semaphore_wait(barrier_sem, N_DEV - 1)
            send_chunk(c)

        own = pltpu.make_async_copy(
            ysend_ref.at[my],
            out_ref.at[pl.ds(my * m_per, m_per), :],
            store_sem,
        )
        own.start()

        for d in range(1, N_DEV):
            src = (my - d) % N_DEV
            recv = pltpu.make_async_remote_copy(
                src_ref=ysend_ref.at[src],
                dst_ref=out_ref.at[pl.ds(src * m_per, m_per), :],
                send_sem=send_sems.at[d - 1],
                recv_sem=recv_sems.at[src],
                device_id=(src,),
                device_id_type=pl.DeviceIdType.MESH,
            )
            recv.wait_recv()

        own.wait()
        for rdma in sends:
            rdma.wait_send()

    out_shape = jax.ShapeDtypeStruct((N_DEV * m_per, blk), jnp.bfloat16)
    return pl.pallas_call(
        body,
        out_shape=out_shape,
        in_specs=[
            pl.BlockSpec(memory_space=pl.ANY),
            pl.BlockSpec(memory_space=pl.ANY),
        ],
        out_specs=pl.BlockSpec(memory_space=pl.ANY),
        scratch_shapes=[
            pltpu.VMEM((m_per, k), jnp.float32),
            pltpu.VMEM((k, n), jnp.float32),
            pltpu.VMEM((N_DEV, m_per, blk), jnp.bfloat16),
            pltpu.SemaphoreType.DMA((N_CHUNKS + 1,)),
            pltpu.SemaphoreType.DMA,
            pltpu.SemaphoreType.DMA((N_DEV - 1,)),
            pltpu.SemaphoreType.DMA((N_DEV,)),
        ],
        compiler_params=pltpu.CompilerParams(collective_id=0),
    )(
        pltpu.with_memory_space_constraint(x, pltpu.MemorySpace.HBM),
        pltpu.with_memory_space_constraint(w_mat, pltpu.MemorySpace.HBM),
    )
